# baseline (speedup 1.0000x reference)
import jax
import jax.numpy as jnp
from jax.experimental import pallas as pl
from jax.experimental.pallas import tpu as pltpu

N = 4096
EMB = 64
TOP_H = 20
ALPHA = 0.5
BETA = 0.5

RB = 512
NB = N // RB


def _sim_block(ublk, ifull):
    return jax.nn.sigmoid(
        jax.lax.dot_general(ublk, ifull, (((1,), (1,)), ((), ())),
                            preferred_element_type=jnp.float32))


def _topsim_block(x_full, xblk, tbl, e0blk, e1blk):
    mu = jnp.mean(x_full)
    xc = x_full - mu
    xb = xblk - mu
    rn_b = 1.0 / (jnp.sqrt(jnp.sum(xb * xb, axis=1, keepdims=True)) + 1e-8)
    sq = xc * xc
    cn = jax.lax.dot_general(
        jnp.ones((1, EMB), jnp.float32), sq, (((1,), (1,)), ((), ())),
        preferred_element_type=jnp.float32)
    rn_c = 1.0 / (jnp.sqrt(cn) + 1e-8)
    sim = jax.lax.dot_general(
        xb, xc, (((1,), (1,)), ((), ())),
        preferred_element_type=jnp.float32) * rn_b * rn_c

    sb = sim.astype(jnp.bfloat16)
    neg = jnp.array(-jnp.inf, jnp.bfloat16)
    m = jnp.max(sb, axis=1, keepdims=True)
    v1 = m
    for _ in range(TOP_H):
        m = jnp.max(jnp.where(sb < m, sb, neg), axis=1, keepdims=True)
    v21 = m

    keep = jnp.logical_and(sb >= v21, sb < v1)
    masked = jnp.where(keep, sim, 0.0)
    semb = jax.lax.dot_general(
        masked, tbl, (((1,), (0,)), ((), ())),
        preferred_element_type=jnp.float32) * (1.0 / TOP_H)
    return (e0blk + e1blk + xblk) * (1.0 / 3.0) + ALPHA * semb


def _fused_kernel(a_ref, uf_ref, if_ref, ub_ref, ib_ref,
                  uout_ref, iout_ref,
                  u1_ref, i1_ref, u2_ref, i2_ref, acc_ref):
    p = pl.program_id(0)
    i = pl.program_id(1)
    row = i * RB

    @pl.when(p == 0)
    def _phase0():
        sim = _sim_block(ub_ref[...], if_ref[...])
        A = a_ref[...] * (0.5 * sim + 0.5)
        u1_ref[pl.ds(row, RB), :] = jax.lax.dot_general(
            A, if_ref[...], (((1,), (0,)), ((), ())),
            preferred_element_type=jnp.float32)
        contrib = jax.lax.dot_general(
            A, ub_ref[...], (((0,), (0,)), ((), ())),
            preferred_element_type=jnp.float32)

        @pl.when(i == 0)
        def _():
            acc_ref[...] = contrib

        @pl.when(i > 0)
        def _():
            acc_ref[...] = acc_ref[...] + contrib

        @pl.when(i == NB - 1)
        def _():
            i1_ref[...] = acc_ref[...]

    @pl.when(p == 1)
    def _phase1():
        sim = _sim_block(ub_ref[...], if_ref[...])
        A = a_ref[...] * (0.5 * sim + 0.5)
        u2_ref[pl.ds(row, RB), :] = jax.lax.dot_general(
            A, i1_ref[...], (((1,), (0,)), ((), ())),
            preferred_element_type=jnp.float32)
        contrib = jax.lax.dot_general(
            A, u1_ref[pl.ds(row, RB), :], (((0,), (0,)), ((), ())),
            preferred_element_type=jnp.float32)

        @pl.when(i == 0)
        def _():
            acc_ref[...] = contrib

        @pl.when(i > 0)
        def _():
            acc_ref[...] = acc_ref[...] + contrib

        @pl.when(i == NB - 1)
        def _():
            i2_ref[...] = acc_ref[...]

    @pl.when(p == 2)
    def _phase2():
        uout_ref[...] = _topsim_block(
            u2_ref[...], u2_ref[pl.ds(row, RB), :], uf_ref[...],
            ub_ref[...], u1_ref[pl.ds(row, RB), :])

    @pl.when(p == 3)
    def _phase3():
        iout_ref[...] = _topsim_block(
            i2_ref[...], i2_ref[pl.ds(row, RB), :], if_ref[...],
            ib_ref[...], i1_ref[pl.ds(row, RB), :])


def kernel(emb_user, emb_item, adj):
    u_out, i_out = pl.pallas_call(
        _fused_kernel,
        grid=(4, NB),
        in_specs=[
            pl.BlockSpec((RB, N), lambda p, i: (jnp.where(p < 2, i, NB - 1), 0)),
            pl.BlockSpec((N, EMB), lambda p, i: (0, 0)),
            pl.BlockSpec((N, EMB), lambda p, i: (0, 0)),
            pl.BlockSpec((RB, EMB), lambda p, i: (jnp.where(p < 3, i, NB - 1), 0)),
            pl.BlockSpec((RB, EMB), lambda p, i: (jnp.where(p == 3, i, 0), 0)),
        ],
        out_specs=[
            pl.BlockSpec((RB, EMB),
                         lambda p, i: (jnp.where(p == 2, i,
                                                 jnp.where(p == 3, NB - 1, 0)), 0)),
            pl.BlockSpec((RB, EMB), lambda p, i: (jnp.where(p == 3, i, 0), 0)),
        ],
        out_shape=[
            jax.ShapeDtypeStruct((N, EMB), jnp.float32),
            jax.ShapeDtypeStruct((N, EMB), jnp.float32),
        ],
        scratch_shapes=[
            pltpu.VMEM((N, EMB), jnp.float32),
            pltpu.VMEM((N, EMB), jnp.float32),
            pltpu.VMEM((N, EMB), jnp.float32),
            pltpu.VMEM((N, EMB), jnp.float32),
            pltpu.VMEM((N, EMB), jnp.float32),
        ],
    )(adj, emb_user, emb_item, emb_user, emb_item)
    return u_out, i_out


# speedup vs baseline: 2.9203x; 2.9203x over previous
import jax
import jax.numpy as jnp
from jax.experimental import pallas as pl
from jax.experimental.pallas import tpu as pltpu

N = 4096
EMB = 64
TOP_H = 20
ALPHA = 0.5
BETA = 0.5

RB = 512
NB = N // RB


def _gcn_kernel(a_ref, uf_ref, if_ref, ub_ref,
                lu_ref, li_ref,
                pu_ref, pi_ref, cu_ref, acc_ref):
    p = pl.program_id(0)
    i = pl.program_id(1)
    row = i * RB

    @pl.when(jnp.logical_and(p == 0, i == 0))
    def _():
        pu_ref[...] = uf_ref[...]
        pi_ref[...] = if_ref[...]

    @pl.when(jnp.logical_and(p == 1, i == 0))
    def _():
        pu_ref[...] = cu_ref[...]
        pi_ref[...] = acc_ref[...]

    sim = jax.nn.sigmoid(
        jax.lax.dot_general(ub_ref[...], if_ref[...], (((1,), (1,)), ((), ())),
                            preferred_element_type=jnp.float32))
    A = a_ref[...] * (0.5 * sim + 0.5)
    nu = jax.lax.dot_general(
        A, pi_ref[...], (((1,), (0,)), ((), ())),
        preferred_element_type=jnp.float32)
    lu_ref[0, :, :] = nu
    cu_ref[pl.ds(row, RB), :] = nu
    contrib = jax.lax.dot_general(
        A, pu_ref[pl.ds(row, RB), :], (((0,), (0,)), ((), ())),
        preferred_element_type=jnp.float32)

    @pl.when(i == 0)
    def _():
        acc_ref[...] = contrib

    @pl.when(i > 0)
    def _():
        acc_ref[...] = acc_ref[...] + contrib

    @pl.when(i == NB - 1)
    def _():
        li_ref[0, :, :] = acc_ref[...]


def _gcn_call(emb_user, emb_item, adj):
    return pl.pallas_call(
        _gcn_kernel,
        grid=(2, NB),
        in_specs=[
            pl.BlockSpec((RB, N), lambda p, i: (i, 0)),
            pl.BlockSpec((N, EMB), lambda p, i: (0, 0)),
            pl.BlockSpec((N, EMB), lambda p, i: (0, 0)),
            pl.BlockSpec((RB, EMB), lambda p, i: (i, 0)),
        ],
        out_specs=[
            pl.BlockSpec((1, RB, EMB), lambda p, i: (p, i, 0)),
            pl.BlockSpec((1, N, EMB), lambda p, i: (p, 0, 0)),
        ],
        out_shape=[
            jax.ShapeDtypeStruct((2, N, EMB), jnp.float32),
            jax.ShapeDtypeStruct((2, N, EMB), jnp.float32),
        ],
        scratch_shapes=[
            pltpu.VMEM((N, EMB), jnp.float32),
            pltpu.VMEM((N, EMB), jnp.float32),
            pltpu.VMEM((N, EMB), jnp.float32),
            pltpu.VMEM((N, EMB), jnp.float32),
        ],
    )(adj, emb_user, emb_item, emb_user)


def _topsim_kernel(xst_ref, tblst_ref, xb_ref, e0_ref, e1_ref, out_ref):
    s = pl.program_id(0)
    half = pl.multiple_of(jnp.where(s < NB, 0, N), N)
    x = xst_ref[pl.ds(half, N), :]
    tbl = tblst_ref[pl.ds(half, N), :]
    mu = jnp.mean(x)
    xc = x - mu
    xb = xb_ref[...] - mu
    rn_b = 1.0 / (jnp.sqrt(jnp.sum(xb * xb, axis=1, keepdims=True)) + 1e-8)
    sq = xc * xc
    cn = jax.lax.dot_general(
        jnp.ones((1, EMB), jnp.float32), sq, (((1,), (1,)), ((), ())),
        preferred_element_type=jnp.float32)
    rn_c = 1.0 / (jnp.sqrt(cn) + 1e-8)
    sim = jax.lax.dot_general(
        xb, xc, (((1,), (1,)), ((), ())),
        preferred_element_type=jnp.float32) * rn_b * rn_c

    sb = sim.astype(jnp.bfloat16)
    neg = jnp.array(-jnp.inf, jnp.bfloat16)
    m = jnp.max(sb, axis=1, keepdims=True)
    v1 = m
    for _ in range(TOP_H):
        m = jnp.max(jnp.where(sb < m, sb, neg), axis=1, keepdims=True)
    v21 = m

    keep = jnp.logical_and(sb >= v21, sb < v1)
    masked = jnp.where(keep, sim, 0.0)
    semb = jax.lax.dot_general(
        masked, tbl, (((1,), (0,)), ((), ())),
        preferred_element_type=jnp.float32) * (1.0 / TOP_H)
    out_ref[...] = (e0_ref[...] + e1_ref[...] + xb_ref[...]) * (1.0 / 3.0) \
        + ALPHA * semb


def _topsim_call(xst, tblst, e1st):
    return pl.pallas_call(
        _topsim_kernel,
        grid=(2 * NB,),
        in_specs=[
            pl.BlockSpec((2 * N, EMB), lambda i: (0, 0)),
            pl.BlockSpec((2 * N, EMB), lambda i: (0, 0)),
            pl.BlockSpec((RB, EMB), lambda i: (i, 0)),
            pl.BlockSpec((RB, EMB), lambda i: (i, 0)),
            pl.BlockSpec((RB, EMB), lambda i: (i, 0)),
        ],
        out_specs=pl.BlockSpec((RB, EMB), lambda i: (i, 0)),
        out_shape=jax.ShapeDtypeStruct((2 * N, EMB), jnp.float32),
    )(xst, tblst, xst, tblst, e1st)


def kernel(emb_user, emb_item, adj):
    lu, li = _gcn_call(emb_user, emb_item, adj)
    xst = jnp.concatenate([lu[1], li[1]], axis=0)
    e1st = jnp.concatenate([lu[0], li[0]], axis=0)
    tblst = jnp.concatenate([emb_user, emb_item], axis=0)
    outst = _topsim_call(xst, tblst, e1st)
    return outst[:N], outst[N:]
